# Initial kernel scaffold; baseline (speedup 1.0000x reference)
import jax
import jax.numpy as jnp
from jax.experimental import pallas as pl


def kernel(inputs, targets):
    raise NotImplementedError("write your pallas kernel here")


# speedup vs baseline: 4.5483x; 4.5483x over previous
import jax
import jax.numpy as jnp
from jax.experimental import pallas as pl
from jax.experimental.pallas import tpu as pltpu

_NCLS = 31
_BG = 30
_BUDGET = 80000.0
_ROWS = 64


def _iou_kernel(x_ref, t_ref, out_ref, acc_ref, cnt_ref):
    b = pl.program_id(0)
    r = pl.program_id(1)
    nb = pl.num_programs(1)
    step = b * nb + r
    nsteps = pl.num_programs(0) * nb

    @pl.when(step == 0)
    def _init():
        acc_ref[0] = 0.0
        acc_ref[1] = 0.0
        cnt_ref[0] = 0

    x = x_ref[0]
    m = x[0]
    idx = jnp.zeros_like(m)
    for c in range(1, _NCLS):
        xc = x[c]
        gt = xc > m
        m = jnp.where(gt, xc, m)
        idx = jnp.where(gt, jnp.float32(c), idx)
    p = idx

    t = t_ref[0]
    tf = t.astype(jnp.float32)
    bg = t == _BG
    bgf = bg.astype(jnp.float32)

    i_nb = jnp.sum(jnp.where(bg, 0.0, p * tf))
    t_nb = jnp.sum(jnp.where(bg, 0.0, p + tf))

    rows, cols = bgf.shape
    jj = jax.lax.broadcasted_iota(jnp.int32, (cols, cols), 0)
    kk = jax.lax.broadcasted_iota(jnp.int32, (cols, cols), 1)
    tri_inc = (jj <= kk).astype(jnp.float32)
    cs_in = jnp.dot(bgf, tri_inc, preferred_element_type=jnp.float32)
    row_tot = cs_in[:, cols - 1:cols]
    ii = jax.lax.broadcasted_iota(jnp.int32, (rows, rows), 0)
    ll = jax.lax.broadcasted_iota(jnp.int32, (rows, rows), 1)
    tri_lo = (ll < ii).astype(jnp.float32)
    r_pref = jnp.dot(tri_lo, row_tot, preferred_element_type=jnp.float32)
    rank_ex = r_pref + (cs_in - bgf)

    offset = cnt_ref[0].astype(jnp.float32)
    include = bg & (offset + rank_ex < _BUDGET)
    s_bg = jnp.sum(jnp.where(include, p, 0.0))
    n_inc = jnp.sum(include.astype(jnp.float32))

    acc_ref[0] += i_nb + jnp.float32(_BG) * s_bg
    acc_ref[1] += t_nb + s_bg + jnp.float32(_BG) * n_inc
    cnt_ref[0] += jnp.sum(bg.astype(jnp.int32))

    @pl.when(step == nsteps - 1)
    def _fin():
        inter = acc_ref[0]
        total = acc_ref[1]
        union = total - inter
        out_ref[0, 0] = 1.0 - (inter + 1.0) / (union + 1.0)


def kernel(inputs, targets):
    b, ncls, h, w = inputs.shape
    nb = h // _ROWS
    out = pl.pallas_call(
        _iou_kernel,
        grid=(b, nb),
        in_specs=[
            pl.BlockSpec((1, ncls, _ROWS, w), lambda i, j: (i, 0, j, 0)),
            pl.BlockSpec((1, _ROWS, w), lambda i, j: (i, j, 0)),
        ],
        out_specs=pl.BlockSpec(
            (1, 1), lambda i, j: (0, 0), memory_space=pltpu.SMEM),
        out_shape=jax.ShapeDtypeStruct((1, 1), jnp.float32),
        scratch_shapes=[
            pltpu.SMEM((2,), jnp.float32),
            pltpu.SMEM((1,), jnp.int32),
        ],
    )(inputs, targets)
    return out[0, 0]


# speedup vs baseline: 5.6780x; 1.2484x over previous
import jax
import jax.numpy as jnp
from jax.experimental import pallas as pl
from jax.experimental.pallas import tpu as pltpu

_NCLS = 31
_BG = 30
_BUDGET = 80000.0
_ROWS = 128


def _iou_kernel(x_ref, t_ref, out_ref, acc_ref, cnt_ref):
    b = pl.program_id(0)
    r = pl.program_id(1)
    nb = pl.num_programs(1)
    step = b * nb + r
    nsteps = pl.num_programs(0) * nb

    @pl.when(step == 0)
    def _init():
        acc_ref[0] = 0.0
        acc_ref[1] = 0.0
        cnt_ref[0] = 0

    x = x_ref[0]
    m = x[0]
    idx = jnp.zeros_like(m)
    for c in range(1, _NCLS):
        xc = x[c]
        gt = xc > m
        m = jnp.where(gt, xc, m)
        idx = jnp.where(gt, jnp.float32(c), idx)
    p = idx

    t = t_ref[0]
    tf = t.astype(jnp.float32)
    bg = t == _BG
    bgf = bg.astype(jnp.float32)

    i_nb = jnp.sum(jnp.where(bg, 0.0, p * tf))
    t_nb = jnp.sum(jnp.where(bg, 0.0, p + tf))

    rows, cols = bgf.shape
    jj = jax.lax.broadcasted_iota(jnp.int32, (cols, cols), 0)
    kk = jax.lax.broadcasted_iota(jnp.int32, (cols, cols), 1)
    tri_inc = (jj <= kk).astype(jnp.float32)
    cs_in = jnp.dot(bgf, tri_inc, preferred_element_type=jnp.float32)
    row_tot = cs_in[:, cols - 1:cols]
    ii = jax.lax.broadcasted_iota(jnp.int32, (rows, rows), 0)
    ll = jax.lax.broadcasted_iota(jnp.int32, (rows, rows), 1)
    tri_lo = (ll < ii).astype(jnp.float32)
    r_pref = jnp.dot(tri_lo, row_tot, preferred_element_type=jnp.float32)
    rank_ex = r_pref + (cs_in - bgf)

    offset = cnt_ref[0].astype(jnp.float32)
    include = bg & (offset + rank_ex < _BUDGET)
    s_bg = jnp.sum(jnp.where(include, p, 0.0))
    n_inc = jnp.sum(include.astype(jnp.float32))

    acc_ref[0] += i_nb + jnp.float32(_BG) * s_bg
    acc_ref[1] += t_nb + s_bg + jnp.float32(_BG) * n_inc
    cnt_ref[0] += jnp.sum(bg.astype(jnp.int32))

    @pl.when(step == nsteps - 1)
    def _fin():
        inter = acc_ref[0]
        total = acc_ref[1]
        union = total - inter
        out_ref[0, 0] = 1.0 - (inter + 1.0) / (union + 1.0)


def kernel(inputs, targets):
    b, ncls, h, w = inputs.shape
    nb = h // _ROWS
    out = pl.pallas_call(
        _iou_kernel,
        grid=(b, nb),
        in_specs=[
            pl.BlockSpec((1, ncls, _ROWS, w), lambda i, j: (i, 0, j, 0)),
            pl.BlockSpec((1, _ROWS, w), lambda i, j: (i, j, 0)),
        ],
        out_specs=pl.BlockSpec(
            (1, 1), lambda i, j: (0, 0), memory_space=pltpu.SMEM),
        out_shape=jax.ShapeDtypeStruct((1, 1), jnp.float32),
        scratch_shapes=[
            pltpu.SMEM((2,), jnp.float32),
            pltpu.SMEM((1,), jnp.int32),
        ],
    )(inputs, targets)
    return out[0, 0]
